# baseline (speedup 1.0000x reference)
import dataclasses
import functools

import jax
import jax.numpy as jnp
from jax.experimental import pallas as pl
from jax.experimental.pallas import tpu as pltpu
from jax.experimental.pallas import tpu_sc as plsc


def _prep_body(cin_ref, r_ref, cbmta_ref, iotac_ref, cs_ref, idx_ref, *,
               m_sub, kc, d_sub, rb1):
    f32, bf16 = jnp.float32, jnp.bfloat16
    i = pl.program_id(0)
    x16 = cin_ref[pl.ds(i * rb1, rb1), :]
    rot = jnp.dot(x16, r_ref[...], preferred_element_type=f32)
    rot16 = rot.astype(bf16)
    cs_ref[...] = rot16
    rot_aug = jnp.concatenate(
        [rot16, jnp.ones((rb1, 1), bf16)], axis=1)
    dims_t = (((1,), (1,)), ((), ()))
    dist_t = jax.lax.dot_general(cbmta_ref[...], rot_aug, dims_t,
                                 preferred_element_type=f32)
    d3 = dist_t.reshape(m_sub, kc, rb1)
    min3 = jnp.min(d3, axis=1, keepdims=True)
    oh_t = (d3 == min3).astype(bf16).reshape(m_sub * kc, rb1)
    dims_c = (((0,), (0,)), ((), ()))
    idxf = jax.lax.dot_general(oh_t, iotac_ref[...], dims_c,
                               preferred_element_type=f32)
    idxk = jnp.minimum(idxf.astype(jnp.int32), kc - 1)
    offs = jax.lax.broadcasted_iota(jnp.int32, (rb1, m_sub), 1) * kc
    idx_ref[...] = idxk + offs


def _loss_body(oq_ref, r_ref, cin_ref, cs_ref, cp_ref, od_ref, op_ref):
    f32, bf16 = jnp.float32, jnp.bfloat16
    oq16 = oq_ref[...]
    rq16 = jnp.dot(oq16, r_ref[...],
                   preferred_element_type=f32).astype(bf16)
    dims = (((1,), (1,)), ((), ()))

    t = jax.lax.dot_general(oq16, cin_ref[...], dims,
                            preferred_element_type=f32)
    mt = jnp.max(t, axis=1, keepdims=True)
    et = jnp.exp(t - mt)
    st = jnp.sum(et, axis=1, keepdims=True)

    def student_term(c_ref, shift):
        s = jax.lax.dot_general(rq16, c_ref[...], dims,
                                preferred_element_type=f32)
        if shift is None:
            shift = jnp.max(s, axis=1, keepdims=True)
        es = jnp.exp(s - shift)
        ss = jnp.sum(es, axis=1, keepdims=True)
        num = jnp.sum(et * jnp.log(es + 1e-6 * ss), axis=1, keepdims=True)
        return jnp.sum(num / st - jnp.log(ss))

    dense_part = student_term(cs_ref, mt)
    pq_part = student_term(cp_ref, None)
    od_ref[...] = jnp.full(od_ref.shape, dense_part, f32)
    op_ref[...] = jnp.full(op_ref.shape, pq_part, f32)


def _sc_gather(cb1d, idx_flat, n_idx, n_cb, d_sub):
    window = 512
    mesh = plsc.VectorSubcoreMesh(core_axis_name="c", subcore_axis_name="s")

    cp = pltpu.CompilerParams()
    if "needs_layout_passes" in pltpu.CompilerParams.__dataclass_fields__:
        cp = dataclasses.replace(cp, needs_layout_passes=False)

    @functools.partial(
        pl.kernel,
        out_type=jax.ShapeDtypeStruct((1, n_idx * d_sub), cb1d.dtype),
        mesh=mesh, compiler_params=cp)
    def gather_kernel(cb_hbm, i_hbm, o_hbm):
        def body(cb_vmem, i_vmem, o_vmem):
            lane = jax.lax.broadcasted_iota(jnp.int32, (16,), 0)

            @pl.loop(0, window, step=16)
            def _(j):
                idx_v = i_vmem[0, pl.ds(j, 16)]
                ebase = idx_v * d_sub
                obase = (j + lane) * d_sub
                for d in range(d_sub):
                    vals = plsc.load_gather(cb_vmem.at[0], [ebase + d])
                    plsc.store_scatter(o_vmem.at[0], [obase + d], vals)

        pltpu.emit_pipeline(
            body,
            grid=(n_idx // window,),
            in_specs=[
                pl.BlockSpec((1, n_cb), index_map=lambda i: (0, 0)),
                pl.BlockSpec((1, window), index_map=lambda i: (0, i)),
            ],
            out_specs=[pl.BlockSpec((1, window * d_sub),
                                    index_map=lambda i: (0, i))],
            core_axis_name=("c", "s"),
            dimension_semantics=(pltpu.PARALLEL,),
        )(cb_hbm, i_hbm, o_hbm)

    return gather_kernel(cb1d, idx_flat)


def kernel(query_token_ids, query_attention_mask, doc_token_ids,
           doc_attention_mask, neg_token_ids, neg_attention_mask,
           origin_q_emb, origin_d_emb, origin_n_emb, doc_ids, neg_ids,
           R, codebook):
    f32, bf16 = jnp.float32, jnp.bfloat16
    b, emb = origin_q_emb.shape
    m_sub, kc, d_sub = codebook.shape
    mk = m_sub * kc
    n2 = 2 * b

    eye = jnp.eye(m_sub, dtype=codebook.dtype)
    cbmt = (eye[:, :, None, None] * codebook[:, None, :, :]) \
        .transpose(0, 2, 1, 3).reshape(mk, emb)
    n2col = jnp.sum(codebook * codebook, axis=-1).reshape(mk, 1)
    cbmta = jnp.concatenate([-2.0 * cbmt, n2col], axis=1)
    kline = jnp.arange(kc, dtype=f32)
    iotac = (eye[:, :, None] * kline[None, None, :]) \
        .transpose(0, 2, 1).reshape(mk, m_sub)

    c_in16 = jnp.concatenate([origin_d_emb, origin_n_emb],
                             axis=0).astype(bf16)
    oq16 = origin_q_emb.astype(bf16)
    r16 = R.astype(bf16)
    iotac16 = iotac.astype(bf16)
    cbmta16 = cbmta.astype(bf16)

    rb1 = min(256, n2)
    g1 = n2 // rb1
    full = lambda shape: pl.BlockSpec(shape, lambda i: tuple(0 for _ in shape))
    c_s, idx4 = pl.pallas_call(
        functools.partial(_prep_body, m_sub=m_sub, kc=kc, d_sub=d_sub,
                          rb1=rb1),
        grid=(g1,),
        in_specs=[
            full((n2, emb)),
            full((emb, emb)),
            full((mk, emb + 1)),
            full((mk, m_sub)),
        ],
        out_specs=[
            pl.BlockSpec((rb1, emb), lambda i: (i, 0)),
            pl.BlockSpec((rb1, m_sub), lambda i: (i, 0)),
        ],
        out_shape=[
            jax.ShapeDtypeStruct((n2, emb), bf16),
            jax.ShapeDtypeStruct((n2, m_sub), jnp.int32),
        ],
        compiler_params=pltpu.CompilerParams(
            dimension_semantics=("arbitrary",)),
    )(c_in16, r16, cbmta16, iotac16)

    n_cb = mk * d_sub
    cb1d = codebook.reshape(1, n_cb)
    gathered = _sc_gather(cb1d, idx4.reshape(1, n2 * m_sub),
                          n2 * m_sub, n_cb, d_sub)
    c_p = gathered.reshape(n2, emb).astype(bf16)

    rb2 = min(256, b)
    g2 = b // rb2
    partials = pl.pallas_call(
        _loss_body,
        grid=(g2,),
        in_specs=[
            pl.BlockSpec((rb2, emb), lambda i: (i, 0)),
            full((emb, emb)),
            full((n2, emb)),
            full((n2, emb)),
            full((n2, emb)),
        ],
        out_specs=[
            pl.BlockSpec((1, 8, 128), lambda i: (i, 0, 0)),
            pl.BlockSpec((1, 8, 128), lambda i: (i, 0, 0)),
        ],
        out_shape=[
            jax.ShapeDtypeStruct((g2, 8, 128), f32),
            jax.ShapeDtypeStruct((g2, 8, 128), f32),
        ],
        compiler_params=pltpu.CompilerParams(
            dimension_semantics=("arbitrary",)),
    )(oq16, r16, c_in16, c_s, c_p)

    dense_loss = -jnp.sum(partials[0][:, 0, 0]) / b
    pq_loss = -jnp.sum(partials[1][:, 0, 0]) / b
    ivf_loss = jnp.asarray(0.0, dtype=f32)
    return (dense_loss, ivf_loss, pq_loss)


# speedup vs baseline: 1.3520x; 1.3520x over previous
import functools

import jax
import jax.numpy as jnp
from jax.experimental import pallas as pl
from jax.experimental.pallas import tpu as pltpu


def _body(oq_ref, cin_ref, r_ref, cbmta_ref, cbmt_ref, od_ref, op_ref,
          cs_ref, cp_ref, *, m_sub, kc, rb1, g1, rb2, g2):
    f32, bf16 = jnp.float32, jnp.bfloat16
    i = pl.program_id(0)

    @pl.when(i < g1)
    def _prep():
        x16 = cin_ref[pl.ds(i * rb1, rb1), :]
        rot = jnp.dot(x16, r_ref[...], preferred_element_type=f32)
        rot16 = rot.astype(bf16)
        cs_ref[pl.ds(i * rb1, rb1), :] = rot16
        rot_aug = jnp.concatenate(
            [rot16, jnp.ones((rb1, 1), bf16)], axis=1)
        dims_t = (((1,), (1,)), ((), ()))
        dist_t = jax.lax.dot_general(cbmta_ref[...], rot_aug, dims_t,
                                     preferred_element_type=f32)
        d3 = dist_t.reshape(m_sub, kc, rb1)
        min3 = jnp.min(d3, axis=1, keepdims=True)
        oh_t = (d3 == min3).astype(bf16).reshape(m_sub * kc, rb1)
        dims_c = (((0,), (0,)), ((), ()))
        qnt = jax.lax.dot_general(oh_t, cbmt_ref[...], dims_c,
                                  preferred_element_type=f32)
        cp_ref[pl.ds(i * rb1, rb1), :] = qnt.astype(bf16)

    @pl.when(i >= g1)
    def _loss():
        li = i - g1
        oq16 = oq_ref[pl.ds(li * rb2, rb2), :]
        rq16 = jnp.dot(oq16, r_ref[...],
                       preferred_element_type=f32).astype(bf16)
        dims = (((1,), (1,)), ((), ()))

        t = jax.lax.dot_general(oq16, cin_ref[...], dims,
                                preferred_element_type=f32)
        mt = jnp.max(t, axis=1, keepdims=True)
        et = jnp.exp(t - mt)
        st = jnp.sum(et, axis=1, keepdims=True)

        def student_term(c_ref, shift):
            s = jax.lax.dot_general(rq16, c_ref[...], dims,
                                    preferred_element_type=f32)
            if shift is None:
                shift = jnp.max(s, axis=1, keepdims=True)
            es = jnp.exp(s - shift)
            ss = jnp.sum(es, axis=1, keepdims=True)
            num = jnp.sum(et * jnp.log(es + 1e-6 * ss), axis=1, keepdims=True)
            return jnp.sum(num / st - jnp.log(ss))

        dense_part = student_term(cs_ref, mt)
        pq_part = student_term(cp_ref, None)
        od_ref[pl.ds(li, 1), :, :] = jnp.full((1, 8, 128), dense_part, f32)
        op_ref[pl.ds(li, 1), :, :] = jnp.full((1, 8, 128), pq_part, f32)


def kernel(query_token_ids, query_attention_mask, doc_token_ids,
           doc_attention_mask, neg_token_ids, neg_attention_mask,
           origin_q_emb, origin_d_emb, origin_n_emb, doc_ids, neg_ids,
           R, codebook):
    f32, bf16 = jnp.float32, jnp.bfloat16
    b, emb = origin_q_emb.shape
    m_sub, kc, d_sub = codebook.shape
    mk = m_sub * kc
    n2 = 2 * b

    eye = jnp.eye(m_sub, dtype=codebook.dtype)
    cbmt = (eye[:, :, None, None] * codebook[:, None, :, :]) \
        .transpose(0, 2, 1, 3).reshape(mk, emb)
    n2col = jnp.sum(codebook * codebook, axis=-1).reshape(mk, 1)
    cbmta = jnp.concatenate([-2.0 * cbmt, n2col], axis=1)

    c_in16 = jnp.concatenate([origin_d_emb, origin_n_emb],
                             axis=0).astype(bf16)
    oq16 = origin_q_emb.astype(bf16)
    r16 = R.astype(bf16)
    cbmt16 = cbmt.astype(bf16)
    cbmta16 = cbmta.astype(bf16)

    rb1 = min(256, n2)
    g1 = n2 // rb1
    rb2 = min(256, b)
    g2 = b // rb2

    full = lambda shape: pl.BlockSpec(shape, lambda i: tuple(0 for _ in shape))
    partials = pl.pallas_call(
        functools.partial(_body, m_sub=m_sub, kc=kc,
                          rb1=rb1, g1=g1, rb2=rb2, g2=g2),
        grid=(g1 + g2,),
        in_specs=[
            full((b, emb)),
            full((n2, emb)),
            full((emb, emb)),
            full((mk, emb + 1)),
            full((mk, emb)),
        ],
        out_specs=[
            full((g2, 8, 128)),
            full((g2, 8, 128)),
        ],
        out_shape=[
            jax.ShapeDtypeStruct((g2, 8, 128), f32),
            jax.ShapeDtypeStruct((g2, 8, 128), f32),
        ],
        scratch_shapes=[
            pltpu.VMEM((n2, emb), bf16),
            pltpu.VMEM((n2, emb), bf16),
        ],
        compiler_params=pltpu.CompilerParams(
            dimension_semantics=("arbitrary",)),
    )(oq16, c_in16, r16, cbmta16, cbmt16)

    dense_loss = -jnp.sum(partials[0][:, 0, 0]) / b
    pq_loss = -jnp.sum(partials[1][:, 0, 0]) / b
    ivf_loss = jnp.asarray(0.0, dtype=f32)
    return (dense_loss, ivf_loss, pq_loss)


# speedup vs baseline: 1.3589x; 1.0051x over previous
import functools

import jax
import jax.numpy as jnp
from jax.experimental import pallas as pl
from jax.experimental.pallas import tpu as pltpu


def _body(oq_ref, cin_ref, r_ref, cbmta_ref, cbmt_ref, od_ref, op_ref,
          cs_ref, cp_ref, *, m_sub, kc, rb1, g1, rb2, g2):
    f32, bf16 = jnp.float32, jnp.bfloat16
    i = pl.program_id(0)

    @pl.when(i < g1)
    def _prep():
        x16 = cin_ref[pl.ds(i * rb1, rb1), :]
        rot = jnp.dot(x16, r_ref[...], preferred_element_type=f32)
        rot16 = rot.astype(bf16)
        cs_ref[pl.ds(i * rb1, rb1), :] = rot16
        rot_aug = jnp.concatenate(
            [rot16, jnp.ones((rb1, 1), bf16)], axis=1)
        dims_t = (((1,), (1,)), ((), ()))
        dist_t = jax.lax.dot_general(cbmta_ref[...], rot_aug, dims_t,
                                     preferred_element_type=f32)
        d3 = dist_t.reshape(m_sub, kc, rb1)
        min3 = jnp.min(d3, axis=1, keepdims=True)
        oh_t = (d3 == min3).astype(bf16).reshape(m_sub * kc, rb1)
        dims_c = (((0,), (0,)), ((), ()))
        qnt = jax.lax.dot_general(oh_t, cbmt_ref[...], dims_c,
                                  preferred_element_type=f32)
        cp_ref[pl.ds(i * rb1, rb1), :] = qnt.astype(bf16)

    @pl.when(i >= g1)
    def _loss():
        li = i - g1
        oq16 = oq_ref[pl.ds(li * rb2, rb2), :]
        rq16 = jnp.dot(oq16, r_ref[...],
                       preferred_element_type=f32).astype(bf16)
        dims = (((1,), (1,)), ((), ()))

        t = jax.lax.dot_general(oq16, cin_ref[...], dims,
                                preferred_element_type=f32)
        mt = jnp.max(t, axis=1, keepdims=True)
        et = jnp.exp(t - mt)
        st = jnp.sum(et, axis=1, keepdims=True)

        s_d = jax.lax.dot_general(rq16, cs_ref[...], dims,
                                  preferred_element_type=f32)
        s_p = jax.lax.dot_general(rq16, cp_ref[...], dims,
                                  preferred_element_type=f32)
        mp = jnp.max(s_p, axis=1, keepdims=True)
        es_d = jnp.exp(s_d - mt)
        es_p = jnp.exp(s_p - mp)
        ss_d = jnp.sum(es_d, axis=1, keepdims=True)
        ss_p = jnp.sum(es_p, axis=1, keepdims=True)
        num_d = jnp.sum(et * jnp.log(es_d + 1e-6 * ss_d),
                        axis=1, keepdims=True)
        num_p = jnp.sum(et * jnp.log(es_p + 1e-6 * ss_p),
                        axis=1, keepdims=True)
        dense_part = jnp.sum(num_d / st - jnp.log(ss_d))
        pq_part = jnp.sum(num_p / st - jnp.log(ss_p))
        od_ref[pl.ds(li, 1), :, :] = jnp.full((1, 8, 128), dense_part, f32)
        op_ref[pl.ds(li, 1), :, :] = jnp.full((1, 8, 128), pq_part, f32)


def kernel(query_token_ids, query_attention_mask, doc_token_ids,
           doc_attention_mask, neg_token_ids, neg_attention_mask,
           origin_q_emb, origin_d_emb, origin_n_emb, doc_ids, neg_ids,
           R, codebook):
    f32, bf16 = jnp.float32, jnp.bfloat16
    b, emb = origin_q_emb.shape
    m_sub, kc, d_sub = codebook.shape
    mk = m_sub * kc
    n2 = 2 * b

    eye = jnp.eye(m_sub, dtype=codebook.dtype)
    cbmt = (eye[:, :, None, None] * codebook[:, None, :, :]) \
        .transpose(0, 2, 1, 3).reshape(mk, emb)
    n2col = jnp.sum(codebook * codebook, axis=-1).reshape(mk, 1)
    cbmta = jnp.concatenate([-2.0 * cbmt, n2col], axis=1)

    c_in16 = jnp.concatenate([origin_d_emb, origin_n_emb],
                             axis=0).astype(bf16)
    oq16 = origin_q_emb.astype(bf16)
    r16 = R.astype(bf16)
    cbmt16 = cbmt.astype(bf16)
    cbmta16 = cbmta.astype(bf16)

    rb1 = min(256, n2)
    g1 = n2 // rb1
    rb2 = min(256, b)
    g2 = b // rb2

    full = lambda shape: pl.BlockSpec(shape, lambda i: tuple(0 for _ in shape))
    partials = pl.pallas_call(
        functools.partial(_body, m_sub=m_sub, kc=kc,
                          rb1=rb1, g1=g1, rb2=rb2, g2=g2),
        grid=(g1 + g2,),
        in_specs=[
            full((b, emb)),
            full((n2, emb)),
            full((emb, emb)),
            full((mk, emb + 1)),
            full((mk, emb)),
        ],
        out_specs=[
            full((g2, 8, 128)),
            full((g2, 8, 128)),
        ],
        out_shape=[
            jax.ShapeDtypeStruct((g2, 8, 128), f32),
            jax.ShapeDtypeStruct((g2, 8, 128), f32),
        ],
        scratch_shapes=[
            pltpu.VMEM((n2, emb), bf16),
            pltpu.VMEM((n2, emb), bf16),
        ],
        compiler_params=pltpu.CompilerParams(
            dimension_semantics=("arbitrary",)),
    )(oq16, c_in16, r16, cbmta16, cbmt16)

    dense_loss = -jnp.sum(partials[0][:, 0, 0]) / b
    pq_loss = -jnp.sum(partials[1][:, 0, 0]) / b
    ivf_loss = jnp.asarray(0.0, dtype=f32)
    return (dense_loss, ivf_loss, pq_loss)
